# Initial kernel scaffold; baseline (speedup 1.0000x reference)
import jax
import jax.numpy as jnp
from jax.experimental import pallas as pl


def kernel(x, t_slot, y, y_t_slot, vecs_use, I_array, cand_table, time_embeddings, Ws1, bs1, Ws2, bs2, Wo1, bo1, Wo2, bo2, Wi1, bi1, Wi2, bi2):
    raise NotImplementedError("write your pallas kernel here")


# speedup vs baseline: 8.3862x; 8.3862x over previous
import math

import jax
import jax.numpy as jnp
from jax import lax
from jax.experimental import pallas as pl
from jax.experimental.pallas import tpu as pltpu
from jax.experimental.pallas import tpu_sc as plsc

F32 = jnp.float32
SEQ, USERS = 50, 1024
T = SEQ * USERS
D = 20
NCENT = 64
NCAND = 64
CD = NCAND * D
NLOC_CAND = NCENT * NCAND
PREC = lax.Precision.HIGHEST


def _dot(a, b):
    return jnp.dot(a, b, precision=PREC)


def _mdot(a, b):
    return jnp.dot(a.astype(jnp.bfloat16), b.astype(jnp.bfloat16),
                   preferred_element_type=jnp.float32)


PADW = 128


def _sc_gather(vecs_pad, xv, i_arr, cand_flat):
    info = plsc.get_sparse_core_info()
    ncores, nsub = info.num_cores, info.num_subcores
    nw = ncores * nsub
    tpw = T // nw
    cpw = NLOC_CAND // nw
    ch = 400
    mesh = plsc.VectorSubcoreMesh(core_axis_name="c", subcore_axis_name="s")

    def body(vecs_hbm, xv_hbm, i_hbm, cf_hbm, xemb_out, cent_out, crows_out,
             idx_v, emb_v, cent_v, cidx_v, crows_v, sem_e, sem_c, sem_r):
        wid = lax.axis_index("s") * ncores + lax.axis_index("c")
        base = wid * tpw
        pltpu.sync_copy(xv_hbm.at[pl.ds(base, tpw)], idx_v)
        cp2 = pltpu.async_copy(i_hbm.at[idx_v], cent_v, sem_c)
        cbase = wid * cpw
        pltpu.sync_copy(cf_hbm.at[pl.ds(cbase, cpw)], cidx_v)
        cp3 = pltpu.async_copy(vecs_hbm.at[cidx_v], crows_v, sem_r)
        for c in range(tpw // ch):
            cp1 = pltpu.async_copy(
                vecs_hbm.at[idx_v.at[pl.ds(c * ch, ch)]], emb_v, sem_e)
            cp1.wait()
            pltpu.sync_copy(emb_v, xemb_out.at[pl.ds(base + c * ch, ch)])
        cp2.wait()
        cp3.wait()
        pltpu.sync_copy(cent_v, cent_out.at[pl.ds(base, tpw)])
        pltpu.sync_copy(crows_v, crows_out.at[pl.ds(cbase, cpw)])

    fn = pl.kernel(
        body,
        out_type=(jax.ShapeDtypeStruct((T, PADW), F32),
                  jax.ShapeDtypeStruct((T,), jnp.int32),
                  jax.ShapeDtypeStruct((NLOC_CAND, PADW), F32)),
        mesh=mesh,
        scratch_types=[pltpu.VMEM((tpw,), jnp.int32),
                       pltpu.VMEM((ch, PADW), F32),
                       pltpu.VMEM((tpw,), jnp.int32),
                       pltpu.VMEM((cpw,), jnp.int32),
                       pltpu.VMEM((cpw, PADW), F32),
                       pltpu.SemaphoreType.DMA,
                       pltpu.SemaphoreType.DMA,
                       pltpu.SemaphoreType.DMA],
    )
    return fn(vecs_pad, xv, i_arr, cand_flat)


def _tables_body(cr, te, w1, b1, w2, b2, out):
    v = cr[...]
    t2 = te[2:3, :]
    h = _mdot(v, w1[0:D, :]) + _mdot(t2, w1[D:2 * D, :]) + b1[...]
    h = jnp.maximum(h, 0.0)
    out[...] = _mdot(h, w2[...]) + b2[...]


def _tc_tables(cand_rows, te, wi1, bi1, wi2, bi2):
    return pl.pallas_call(
        _tables_body,
        out_shape=jax.ShapeDtypeStruct((NLOC_CAND, D), F32),
    )(cand_rows, te, wi1, bi1.reshape(1, -1), wi2, bi2.reshape(1, -1))


_EULER = math.e


def _main_body(hist_ref, xe_ref, ts_ref, ct_ref, s2_ref, r2_ref,
               ws1, bs1, ws2, bs2, wo1, bo1, wo2, bo2, te, out_ref):
    hf = hist_ref[0]
    xe = xe_ref[0][:, 0:D]
    tsl = ts_ref[0, 0]
    ct = ct_ref[0, 0]

    h1 = jnp.maximum(_mdot(hf, ws1[...]) + bs1[...], 0.0)
    hseq = _mdot(h1, ws2[...]) + bs2[...]

    hh = tsl % 24
    m0 = (hh >= 22) | (hh < 6)
    m1 = (hh >= 6) & (hh < 14)
    m2 = jnp.logical_not(m0 | m1)
    tep = _mdot(te[0:3, :], wo1[D:2 * D, :])
    tec = (m0.astype(F32)[:, None] * tep[0:1, :]
           + m1.astype(F32)[:, None] * tep[1:2, :]
           + m2.astype(F32)[:, None] * tep[2:3, :])
    x1 = jnp.maximum(_mdot(xe, wo1[0:D, :]) + tec + bo1[...], 0.0)
    xi = _mdot(x1, wo2[...]) + bo2[...]

    q = (hseq + xi) * 0.5

    iota64 = lax.broadcasted_iota(jnp.int32, (USERS, NCAND), 1)
    onehot = (ct[:, None] == iota64).astype(F32)
    cand_s = _dot(onehot, s2_ref[...])
    cand_r = _dot(onehot, r2_ref[...])

    tile_m = (lax.broadcasted_iota(jnp.int32, (D, CD), 1) % D
              == lax.broadcasted_iota(jnp.int32, (D, CD), 0)).astype(F32)
    qt = _dot(q, tile_m)
    diff = qt - cand_s
    rsum = (lax.broadcasted_iota(jnp.int32, (CD, NCAND), 0) // D
            == lax.broadcasted_iota(jnp.int32, (CD, NCAND), 1)).astype(F32)
    d2 = _dot(diff * diff, rsum)
    score = jnp.exp(-0.02 * jnp.sqrt(d2 + 1e-12))

    neg = F32(-1e30)
    alive = jnp.ones((USERS, NCAND), dtype=jnp.bool_)
    sel = jnp.zeros((USERS, NCAND), dtype=jnp.bool_)
    for _ in range(10):
        cur = jnp.where(alive, score, neg)
        m = jnp.max(cur, axis=1, keepdims=True)
        first = jnp.min(jnp.where(cur == m, iota64, NCAND),
                        axis=1, keepdims=True)
        pick = iota64 == first
        sel = sel | pick
        alive = alive & jnp.logical_not(pick)

    wt = jnp.where(sel, jnp.exp(score), 0.0)
    z = jnp.sum(wt, axis=1, keepdims=True) + _EULER
    rexp = (lax.broadcasted_iota(jnp.int32, (NCAND, CD), 1) // D
            == lax.broadcasted_iota(jnp.int32, (NCAND, CD), 0)).astype(F32)
    wrep = _dot(wt, rexp)
    tile_t = (lax.broadcasted_iota(jnp.int32, (CD, D), 0) % D
              == lax.broadcasted_iota(jnp.int32, (CD, D), 1)).astype(F32)
    outn = _dot(wrep * cand_r, tile_t)
    out_ref[0] = (outn + _EULER * xe) / z


def _tc_main(histf, x3, ts3, ct3, s2, r2,
             ws1, bs1, ws2, bs2, wo1, bo1, wo2, bo2, te):
    bs1, bs2 = bs1.reshape(1, -1), bs2.reshape(1, -1)
    bo1, bo2 = bo1.reshape(1, -1), bo2.reshape(1, -1)
    full2 = lambda a: pl.BlockSpec(a.shape, lambda s: (0, 0))
    in_specs = [
        pl.BlockSpec((1, USERS, 100), lambda s: (s, 0, 0)),
        pl.BlockSpec((1, USERS, PADW), lambda s: (s, 0, 0)),
        pl.BlockSpec((1, 1, USERS), lambda s: (s, 0, 0)),
        pl.BlockSpec((1, 1, USERS), lambda s: (s, 0, 0)),
        full2(s2), full2(r2),
        full2(ws1), full2(bs1), full2(ws2), full2(bs2),
        full2(wo1), full2(bo1), full2(wo2), full2(bo2), full2(te),
    ]
    return pl.pallas_call(
        _main_body,
        grid=(SEQ,),
        in_specs=in_specs,
        out_specs=pl.BlockSpec((1, USERS, D), lambda s: (s, 0, 0)),
        out_shape=jax.ShapeDtypeStruct((SEQ, USERS, D), F32),
    )(histf, x3, ts3, ct3, s2, r2,
      ws1, bs1, ws2, bs2, wo1, bo1, wo2, bo2, te)


def kernel(x, t_slot, y, y_t_slot, vecs_use, I_array, cand_table,
           time_embeddings, Ws1, bs1, Ws2, bs2, Wo1, bo1, Wo2, bo2,
           Wi1, bi1, Wi2, bi2):
    xv = x.reshape(-1).astype(jnp.int32)
    i_arr = I_array.astype(jnp.int32)
    cf = cand_table.reshape(-1).astype(jnp.int32)
    vecs_pad = jnp.pad(vecs_use, ((0, 0), (0, PADW - D)))

    x_emb_pad, cent, cand_rows_pad = _sc_gather(vecs_pad, xv, i_arr, cf)
    cand_rows = cand_rows_pad[:, :D]

    s_tab = _tc_tables(cand_rows, time_embeddings, Wi1, bi1, Wi2, bi2)
    s2 = s_tab.reshape(NCENT, CD)
    r2 = cand_rows.reshape(NCENT, CD)

    xp3 = x_emb_pad.reshape(SEQ, USERS, PADW)
    x3 = xp3[:, :, :D]

    def shift(k):
        return jnp.concatenate([x3[0:k], x3[0:SEQ - k]], axis=0)

    histf = jnp.concatenate(
        [shift(4), shift(3), shift(2), shift(1), x3], axis=-1)
    ts3 = t_slot.reshape(SEQ, 1, USERS).astype(jnp.int32)
    ct3 = cent.reshape(SEQ, 1, USERS)

    out = _tc_main(histf, xp3, ts3, ct3, s2, r2,
                   Ws1, bs1, Ws2, bs2, Wo1, bo1, Wo2, bo2, time_embeddings)
    return out.reshape(T, D)


# speedup vs baseline: 14.0699x; 1.6777x over previous
import math

import jax
import jax.numpy as jnp
from jax import lax
from jax.experimental import pallas as pl
from jax.experimental.pallas import tpu as pltpu
from jax.experimental.pallas import tpu_sc as plsc

F32 = jnp.float32
SEQ, USERS = 50, 1024
T = SEQ * USERS
D = 20
NCENT = 64
NCAND = 64
CD = NCAND * D
NLOC_CAND = NCENT * NCAND
PREC = lax.Precision.HIGHEST


def _dot(a, b):
    return jnp.dot(a, b, precision=PREC)


BF16 = jnp.bfloat16


def _bdot(a16, b16):
    return jnp.dot(a16, b16, preferred_element_type=F32)


def _split3(x):
    h1 = x.astype(BF16)
    r1 = x - h1.astype(F32)
    h2 = r1.astype(BF16)
    r2 = r1 - h2.astype(F32)
    h3 = r2.astype(BF16)
    return h1, h2, h3


def _dotA01(a, b16):
    return sum(_bdot(h, b16) for h in _split3(a))


def _dot01B(a16, b):
    return sum(_bdot(a16, h) for h in _split3(b))


def _mdot(a, b):
    return jnp.dot(a.astype(jnp.bfloat16), b.astype(jnp.bfloat16),
                   preferred_element_type=jnp.float32)


PADW = 128


def _sc_gather(vecs_pad, xv, i_arr, cand_flat):
    info = plsc.get_sparse_core_info()
    ncores, nsub = info.num_cores, info.num_subcores
    nw = ncores * nsub
    tpw = T // nw
    cpw = NLOC_CAND // nw
    ch = 400
    mesh = plsc.VectorSubcoreMesh(core_axis_name="c", subcore_axis_name="s")

    def body(vecs_hbm, xv_hbm, i_hbm, cf_hbm, xemb_out, cent_out, crows_out,
             idx_v, emb_v, cent_v, cidx_v, crows_v, sem_e, sem_c, sem_r):
        wid = lax.axis_index("s") * ncores + lax.axis_index("c")
        base = wid * tpw
        pltpu.sync_copy(xv_hbm.at[pl.ds(base, tpw)], idx_v)
        cp2 = pltpu.async_copy(i_hbm.at[idx_v], cent_v, sem_c)
        cbase = wid * cpw
        pltpu.sync_copy(cf_hbm.at[pl.ds(cbase, cpw)], cidx_v)
        cp3 = pltpu.async_copy(vecs_hbm.at[cidx_v], crows_v, sem_r)
        for c in range(tpw // ch):
            cp1 = pltpu.async_copy(
                vecs_hbm.at[idx_v.at[pl.ds(c * ch, ch)]], emb_v, sem_e)
            cp1.wait()
            pltpu.sync_copy(emb_v, xemb_out.at[pl.ds(base + c * ch, ch)])
        cp2.wait()
        cp3.wait()
        pltpu.sync_copy(cent_v, cent_out.at[pl.ds(base, tpw)])
        pltpu.sync_copy(crows_v, crows_out.at[pl.ds(cbase, cpw)])

    fn = pl.kernel(
        body,
        out_type=(jax.ShapeDtypeStruct((T, PADW), F32),
                  jax.ShapeDtypeStruct((T,), jnp.int32),
                  jax.ShapeDtypeStruct((NLOC_CAND, PADW), F32)),
        mesh=mesh,
        scratch_types=[pltpu.VMEM((tpw,), jnp.int32),
                       pltpu.VMEM((ch, PADW), F32),
                       pltpu.VMEM((tpw,), jnp.int32),
                       pltpu.VMEM((cpw,), jnp.int32),
                       pltpu.VMEM((cpw, PADW), F32),
                       pltpu.SemaphoreType.DMA,
                       pltpu.SemaphoreType.DMA,
                       pltpu.SemaphoreType.DMA],
    )
    return fn(vecs_pad, xv, i_arr, cand_flat)


def _tables_body(cr, te, w1, b1, w2, b2, out):
    v = cr[...]
    t2 = te[2:3, :]
    h = _mdot(v, w1[0:D, :]) + _mdot(t2, w1[D:2 * D, :]) + b1[...]
    h = jnp.maximum(h, 0.0)
    out[...] = _mdot(h, w2[...]) + b2[...]


def _tc_tables(cand_rows, te, wi1, bi1, wi2, bi2):
    return pl.pallas_call(
        _tables_body,
        out_shape=jax.ShapeDtypeStruct((NLOC_CAND, D), F32),
    )(cand_rows, te, wi1, bi1.reshape(1, -1), wi2, bi2.reshape(1, -1))


_EULER = math.e


def _main_body(hist_ref, xe_ref, ts_ref, ct_ref, s2_ref, r2_ref,
               ws1, bs1, ws2, bs2, wo1, bo1, wo2, bo2, te, out_ref):
    hf = hist_ref[0]
    xe = xe_ref[0][:, 0:D]
    tsl = ts_ref[0, 0]
    ct = ct_ref[0, 0]

    h1 = jnp.maximum(_mdot(hf, ws1[...]) + bs1[...], 0.0)
    hseq = _mdot(h1, ws2[...]) + bs2[...]

    hh = tsl % 24
    m0 = (hh >= 22) | (hh < 6)
    m1 = (hh >= 6) & (hh < 14)
    m2 = jnp.logical_not(m0 | m1)
    tep = _mdot(te[0:3, :], wo1[D:2 * D, :])
    tec = (m0.astype(F32)[:, None] * tep[0:1, :]
           + m1.astype(F32)[:, None] * tep[1:2, :]
           + m2.astype(F32)[:, None] * tep[2:3, :])
    x1 = jnp.maximum(_mdot(xe, wo1[0:D, :]) + tec + bo1[...], 0.0)
    xi = _mdot(x1, wo2[...]) + bo2[...]

    q = (hseq + xi) * 0.5

    iota64 = lax.broadcasted_iota(jnp.int32, (USERS, NCAND), 1)
    oh16 = (ct[:, None] == iota64).astype(BF16)
    cand_s = _dot01B(oh16, s2_ref[...])
    cand_r = _dot01B(oh16, r2_ref[...])

    tile16 = (lax.broadcasted_iota(jnp.int32, (D, CD), 1) % D
              == lax.broadcasted_iota(jnp.int32, (D, CD), 0)).astype(BF16)
    qt = _dotA01(q, tile16)
    diff = qt - cand_s
    rsum16 = (lax.broadcasted_iota(jnp.int32, (CD, NCAND), 0) // D
              == lax.broadcasted_iota(jnp.int32, (CD, NCAND), 1)).astype(BF16)
    d2 = _dotA01(diff * diff, rsum16)
    score = jnp.exp(-0.02 * jnp.sqrt(d2 + 1e-12))

    neg = F32(-1e30)
    alive = jnp.ones((USERS, NCAND), dtype=jnp.bool_)
    sel = jnp.zeros((USERS, NCAND), dtype=jnp.bool_)
    for _ in range(10):
        cur = jnp.where(alive, score, neg)
        m = jnp.max(cur, axis=1, keepdims=True)
        first = jnp.min(jnp.where(cur == m, iota64, NCAND),
                        axis=1, keepdims=True)
        pick = iota64 == first
        sel = sel | pick
        alive = alive & jnp.logical_not(pick)

    wt = jnp.where(sel, jnp.exp(score), 0.0)
    z = jnp.sum(wt, axis=1, keepdims=True) + _EULER
    rexp16 = (lax.broadcasted_iota(jnp.int32, (NCAND, CD), 1) // D
              == lax.broadcasted_iota(jnp.int32, (NCAND, CD), 0)).astype(BF16)
    wrep = _dotA01(wt, rexp16)
    tile_t16 = (lax.broadcasted_iota(jnp.int32, (CD, D), 0) % D
                == lax.broadcasted_iota(jnp.int32, (CD, D), 1)).astype(BF16)
    outn = _dotA01(wrep * cand_r, tile_t16)
    out_ref[0] = (outn + _EULER * xe) / z


def _tc_main(histf, x3, ts3, ct3, s2, r2,
             ws1, bs1, ws2, bs2, wo1, bo1, wo2, bo2, te):
    bs1, bs2 = bs1.reshape(1, -1), bs2.reshape(1, -1)
    bo1, bo2 = bo1.reshape(1, -1), bo2.reshape(1, -1)
    full2 = lambda a: pl.BlockSpec(a.shape, lambda s: (0, 0))
    in_specs = [
        pl.BlockSpec((1, USERS, 100), lambda s: (s, 0, 0)),
        pl.BlockSpec((1, USERS, PADW), lambda s: (s, 0, 0)),
        pl.BlockSpec((1, 1, USERS), lambda s: (s, 0, 0)),
        pl.BlockSpec((1, 1, USERS), lambda s: (s, 0, 0)),
        full2(s2), full2(r2),
        full2(ws1), full2(bs1), full2(ws2), full2(bs2),
        full2(wo1), full2(bo1), full2(wo2), full2(bo2), full2(te),
    ]
    return pl.pallas_call(
        _main_body,
        grid=(SEQ,),
        in_specs=in_specs,
        out_specs=pl.BlockSpec((1, USERS, D), lambda s: (s, 0, 0)),
        out_shape=jax.ShapeDtypeStruct((SEQ, USERS, D), F32),
    )(histf, x3, ts3, ct3, s2, r2,
      ws1, bs1, ws2, bs2, wo1, bo1, wo2, bo2, te)


def kernel(x, t_slot, y, y_t_slot, vecs_use, I_array, cand_table,
           time_embeddings, Ws1, bs1, Ws2, bs2, Wo1, bo1, Wo2, bo2,
           Wi1, bi1, Wi2, bi2):
    xv = x.reshape(-1).astype(jnp.int32)
    i_arr = I_array.astype(jnp.int32)
    cf = cand_table.reshape(-1).astype(jnp.int32)
    vecs_pad = jnp.pad(vecs_use, ((0, 0), (0, PADW - D)))

    x_emb_pad, cent, cand_rows_pad = _sc_gather(vecs_pad, xv, i_arr, cf)
    cand_rows = cand_rows_pad[:, :D]

    s_tab = _tc_tables(cand_rows, time_embeddings, Wi1, bi1, Wi2, bi2)
    s2 = s_tab.reshape(NCENT, CD)
    r2 = cand_rows.reshape(NCENT, CD)

    xp3 = x_emb_pad.reshape(SEQ, USERS, PADW)
    x3 = xp3[:, :, :D]

    def shift(k):
        return jnp.concatenate([x3[0:k], x3[0:SEQ - k]], axis=0)

    histf = jnp.concatenate(
        [shift(4), shift(3), shift(2), shift(1), x3], axis=-1)
    ts3 = t_slot.reshape(SEQ, 1, USERS).astype(jnp.int32)
    ct3 = cent.reshape(SEQ, 1, USERS)

    out = _tc_main(histf, xp3, ts3, ct3, s2, r2,
                   Ws1, bs1, Ws2, bs2, Wo1, bo1, Wo2, bo2, time_embeddings)
    return out.reshape(T, D)


# speedup vs baseline: 17.2742x; 1.2277x over previous
import math

import jax
import jax.numpy as jnp
from jax import lax
from jax.experimental import pallas as pl
from jax.experimental.pallas import tpu as pltpu
from jax.experimental.pallas import tpu_sc as plsc

F32 = jnp.float32
SEQ, USERS = 50, 1024
T = SEQ * USERS
D = 20
NCENT = 64
NCAND = 64
CD = NCAND * D
NLOC_CAND = NCENT * NCAND
PREC = lax.Precision.HIGHEST


def _dot(a, b):
    return jnp.dot(a, b, precision=PREC)


BF16 = jnp.bfloat16


def _bdot(a16, b16):
    return jnp.dot(a16, b16, preferred_element_type=F32)


def _split3(x):
    h1 = x.astype(BF16)
    r1 = x - h1.astype(F32)
    h2 = r1.astype(BF16)
    r2 = r1 - h2.astype(F32)
    h3 = r2.astype(BF16)
    return h1, h2, h3


def _dotA01(a, b16):
    return sum(_bdot(h, b16) for h in _split3(a))


def _dot01B(a16, b):
    return sum(_bdot(a16, h) for h in _split3(b))


def _split2(x):
    h1 = x.astype(BF16)
    h2 = (x - h1.astype(F32)).astype(BF16)
    return h1, h2


def _dotA01_2(a, b16):
    return sum(_bdot(h, b16) for h in _split2(a))


def _dot01B_2(a16, b):
    return sum(_bdot(a16, h) for h in _split2(b))


def _mdot(a, b):
    return jnp.dot(a.astype(jnp.bfloat16), b.astype(jnp.bfloat16),
                   preferred_element_type=jnp.float32)


PADW = 128


def _sc_gather(vecs_pad, xv, i_arr, cand_flat):
    info = plsc.get_sparse_core_info()
    ncores, nsub = info.num_cores, info.num_subcores
    nw = ncores * nsub
    tpw = T // nw
    cpw = NLOC_CAND // nw
    ch = 400
    mesh = plsc.VectorSubcoreMesh(core_axis_name="c", subcore_axis_name="s")

    def body(vecs_hbm, xv_hbm, i_hbm, cf_hbm, xemb_out, cent_out, crows_out,
             idx_v, emb_v, cent_v, cidx_v, crows_v, sem_e, sem_c, sem_r):
        wid = lax.axis_index("s") * ncores + lax.axis_index("c")
        base = wid * tpw
        pltpu.sync_copy(xv_hbm.at[pl.ds(base, tpw)], idx_v)
        cp2 = pltpu.async_copy(i_hbm.at[idx_v], cent_v, sem_c)
        cbase = wid * cpw
        pltpu.sync_copy(cf_hbm.at[pl.ds(cbase, cpw)], cidx_v)
        cp3 = pltpu.async_copy(vecs_hbm.at[cidx_v], crows_v, sem_r)
        for c in range(tpw // ch):
            cp1 = pltpu.async_copy(
                vecs_hbm.at[idx_v.at[pl.ds(c * ch, ch)]], emb_v, sem_e)
            cp1.wait()
            pltpu.sync_copy(emb_v, xemb_out.at[pl.ds(base + c * ch, ch)])
        cp2.wait()
        cp3.wait()
        pltpu.sync_copy(cent_v, cent_out.at[pl.ds(base, tpw)])
        pltpu.sync_copy(crows_v, crows_out.at[pl.ds(cbase, cpw)])

    fn = pl.kernel(
        body,
        out_type=(jax.ShapeDtypeStruct((T, PADW), F32),
                  jax.ShapeDtypeStruct((T,), jnp.int32),
                  jax.ShapeDtypeStruct((NLOC_CAND, PADW), F32)),
        mesh=mesh,
        scratch_types=[pltpu.VMEM((tpw,), jnp.int32),
                       pltpu.VMEM((ch, PADW), F32),
                       pltpu.VMEM((tpw,), jnp.int32),
                       pltpu.VMEM((cpw,), jnp.int32),
                       pltpu.VMEM((cpw, PADW), F32),
                       pltpu.SemaphoreType.DMA,
                       pltpu.SemaphoreType.DMA,
                       pltpu.SemaphoreType.DMA],
    )
    return fn(vecs_pad, xv, i_arr, cand_flat)


def _tables_body(cr, te, w1, b1, w2, b2, out, ns_out):
    v = cr[...]
    t2 = te[2:3, :]
    h = _mdot(v, w1[0:D, :]) + _mdot(t2, w1[D:2 * D, :]) + b1[...]
    h = jnp.maximum(h, 0.0)
    s = _mdot(h, w2[...]) + b2[...]
    out[...] = s
    ns_out[...] = jnp.sum(s * s, axis=1, keepdims=True)


def _tc_tables(cand_rows, te, wi1, bi1, wi2, bi2):
    return pl.pallas_call(
        _tables_body,
        out_shape=(jax.ShapeDtypeStruct((NLOC_CAND, D), F32),
                   jax.ShapeDtypeStruct((NLOC_CAND, 1), F32)),
    )(cand_rows, te, wi1, bi1.reshape(1, -1), wi2, bi2.reshape(1, -1))


_EULER = math.e


def _main_body(hist_ref, xe_ref, ts_ref, ct_ref, s2_ref, ns_ref, r2_ref,
               ws1, bs1, ws2, bs2, wo1, bo1, wo2, bo2, te, out_ref):
    hf = hist_ref[0]
    xe = xe_ref[0][:, 0:D]
    tsl = ts_ref[0, 0]
    ct = ct_ref[0, 0]

    h1 = jnp.maximum(_mdot(hf, ws1[...]) + bs1[...], 0.0)
    hseq = _mdot(h1, ws2[...]) + bs2[...]

    hh = tsl % 24
    m0 = (hh >= 22) | (hh < 6)
    m1 = (hh >= 6) & (hh < 14)
    m2 = jnp.logical_not(m0 | m1)
    tep = _mdot(te[0:3, :], wo1[D:2 * D, :])
    tec = (m0.astype(F32)[:, None] * tep[0:1, :]
           + m1.astype(F32)[:, None] * tep[1:2, :]
           + m2.astype(F32)[:, None] * tep[2:3, :])
    x1 = jnp.maximum(_mdot(xe, wo1[0:D, :]) + tec + bo1[...], 0.0)
    xi = _mdot(x1, wo2[...]) + bo2[...]

    q = (hseq + xi) * 0.5

    iota64 = lax.broadcasted_iota(jnp.int32, (USERS, NCAND), 1)
    oh16 = (ct[:, None] == iota64).astype(BF16)
    cand_sd = _dot01B(oh16, s2_ref[...])
    nc = _dot01B(oh16, ns_ref[...])
    cand_r = _dot01B_2(oh16, r2_ref[...])

    qc = q[:, 0:1] * cand_sd[:, 0:NCAND]
    for dd in range(1, D):
        qc = qc + q[:, dd:dd + 1] * cand_sd[:, dd * NCAND:(dd + 1) * NCAND]
    q2 = jnp.sum(q * q, axis=1, keepdims=True)
    d2 = jnp.maximum(q2 - 2.0 * qc + nc, 0.0)
    score = jnp.exp(-0.02 * jnp.sqrt(d2 + 1e-12))

    neg = F32(-1e30)
    alive = jnp.ones((USERS, NCAND), dtype=jnp.bool_)
    sel = jnp.zeros((USERS, NCAND), dtype=jnp.bool_)
    for _ in range(10):
        cur = jnp.where(alive, score, neg)
        m = jnp.max(cur, axis=1, keepdims=True)
        first = jnp.min(jnp.where(cur == m, iota64, NCAND),
                        axis=1, keepdims=True)
        pick = iota64 == first
        sel = sel | pick
        alive = alive & jnp.logical_not(pick)

    wt = jnp.where(sel, jnp.exp(score), 0.0)
    z = jnp.sum(wt, axis=1, keepdims=True) + _EULER
    rexp16 = (lax.broadcasted_iota(jnp.int32, (NCAND, CD), 1) // D
              == lax.broadcasted_iota(jnp.int32, (NCAND, CD), 0)).astype(BF16)
    wrep = _dotA01_2(wt, rexp16)
    tile_t16 = (lax.broadcasted_iota(jnp.int32, (CD, D), 0) % D
                == lax.broadcasted_iota(jnp.int32, (CD, D), 1)).astype(BF16)
    outn = _dotA01_2(wrep * cand_r, tile_t16)
    out_ref[0] = (outn + _EULER * xe) / z


def _tc_main(histf, x3, ts3, ct3, s2, ns2, r2,
             ws1, bs1, ws2, bs2, wo1, bo1, wo2, bo2, te):
    bs1, bs2 = bs1.reshape(1, -1), bs2.reshape(1, -1)
    bo1, bo2 = bo1.reshape(1, -1), bo2.reshape(1, -1)
    full2 = lambda a: pl.BlockSpec(a.shape, lambda s: (0, 0))
    in_specs = [
        pl.BlockSpec((1, USERS, 100), lambda s: (s, 0, 0)),
        pl.BlockSpec((1, USERS, PADW), lambda s: (s, 0, 0)),
        pl.BlockSpec((1, 1, USERS), lambda s: (s, 0, 0)),
        pl.BlockSpec((1, 1, USERS), lambda s: (s, 0, 0)),
        full2(s2), full2(ns2), full2(r2),
        full2(ws1), full2(bs1), full2(ws2), full2(bs2),
        full2(wo1), full2(bo1), full2(wo2), full2(bo2), full2(te),
    ]
    return pl.pallas_call(
        _main_body,
        grid=(SEQ,),
        in_specs=in_specs,
        out_specs=pl.BlockSpec((1, USERS, D), lambda s: (s, 0, 0)),
        out_shape=jax.ShapeDtypeStruct((SEQ, USERS, D), F32),
    )(histf, x3, ts3, ct3, s2, ns2, r2,
      ws1, bs1, ws2, bs2, wo1, bo1, wo2, bo2, te)


def kernel(x, t_slot, y, y_t_slot, vecs_use, I_array, cand_table,
           time_embeddings, Ws1, bs1, Ws2, bs2, Wo1, bo1, Wo2, bo2,
           Wi1, bi1, Wi2, bi2):
    xv = x.reshape(-1).astype(jnp.int32)
    i_arr = I_array.astype(jnp.int32)
    cf = cand_table.reshape(-1).astype(jnp.int32)
    vecs_pad = jnp.pad(vecs_use, ((0, 0), (0, PADW - D)))

    x_emb_pad, cent, cand_rows_pad = _sc_gather(vecs_pad, xv, i_arr, cf)
    cand_rows = cand_rows_pad[:, :D]

    s_tab, ns = _tc_tables(cand_rows, time_embeddings, Wi1, bi1, Wi2, bi2)
    s2 = s_tab.reshape(NCENT, NCAND, D).transpose(0, 2, 1).reshape(NCENT, CD)
    ns2 = ns.reshape(NCENT, NCAND)
    r2 = cand_rows.reshape(NCENT, CD)

    xp3 = x_emb_pad.reshape(SEQ, USERS, PADW)
    x3 = xp3[:, :, :D]

    def shift(k):
        return jnp.concatenate([x3[0:k], x3[0:SEQ - k]], axis=0)

    histf = jnp.concatenate(
        [shift(4), shift(3), shift(2), shift(1), x3], axis=-1)
    ts3 = t_slot.reshape(SEQ, 1, USERS).astype(jnp.int32)
    ct3 = cent.reshape(SEQ, 1, USERS)

    out = _tc_main(histf, xp3, ts3, ct3, s2, ns2, r2,
                   Ws1, bs1, Ws2, bs2, Wo1, bo1, Wo2, bo2, time_embeddings)
    return out.reshape(T, D)


# speedup vs baseline: 21.2805x; 1.2319x over previous
import math

import jax
import jax.numpy as jnp
from jax import lax
from jax.experimental import pallas as pl
from jax.experimental.pallas import tpu as pltpu
from jax.experimental.pallas import tpu_sc as plsc

F32 = jnp.float32
SEQ, USERS = 50, 1024
T = SEQ * USERS
D = 20
NCENT = 64
NCAND = 64
CD = NCAND * D
NLOC_CAND = NCENT * NCAND
PREC = lax.Precision.HIGHEST


def _dot(a, b):
    return jnp.dot(a, b, precision=PREC)


BF16 = jnp.bfloat16


def _bdot(a16, b16):
    return jnp.dot(a16, b16, preferred_element_type=F32)


def _split3(x):
    h1 = x.astype(BF16)
    r1 = x - h1.astype(F32)
    h2 = r1.astype(BF16)
    r2 = r1 - h2.astype(F32)
    h3 = r2.astype(BF16)
    return h1, h2, h3


def _dotA01(a, b16):
    return sum(_bdot(h, b16) for h in _split3(a))


def _dot01B(a16, b):
    return sum(_bdot(a16, h) for h in _split3(b))


def _split2(x):
    h1 = x.astype(BF16)
    h2 = (x - h1.astype(F32)).astype(BF16)
    return h1, h2


def _dotA01_2(a, b16):
    return sum(_bdot(h, b16) for h in _split2(a))


def _dot01B_2(a16, b):
    return sum(_bdot(a16, h) for h in _split2(b))


def _mdot(a, b):
    return jnp.dot(a.astype(jnp.bfloat16), b.astype(jnp.bfloat16),
                   preferred_element_type=jnp.float32)


PADW = 128


def _sc_gather(vecs_pad, xv, i_arr, cand_flat):
    info = plsc.get_sparse_core_info()
    ncores, nsub = info.num_cores, info.num_subcores
    nw = ncores * nsub
    tpw = T // nw
    cpw = NLOC_CAND // nw
    ch = 400
    mesh = plsc.VectorSubcoreMesh(core_axis_name="c", subcore_axis_name="s")

    def body(vecs_hbm, xv_hbm, i_hbm, cf_hbm, xemb_out, cent_out, crows_out,
             idx_v, emb_v, cent_v, cidx_v, crows_v, sem_e, sem_c, sem_r):
        wid = lax.axis_index("s") * ncores + lax.axis_index("c")
        base = wid * tpw
        pltpu.sync_copy(xv_hbm.at[pl.ds(base, tpw)], idx_v)
        cp2 = pltpu.async_copy(i_hbm.at[idx_v], cent_v, sem_c)
        cbase = wid * cpw
        pltpu.sync_copy(cf_hbm.at[pl.ds(cbase, cpw)], cidx_v)
        cp3 = pltpu.async_copy(vecs_hbm.at[cidx_v], crows_v, sem_r)
        for c in range(tpw // ch):
            cp1 = pltpu.async_copy(
                vecs_hbm.at[idx_v.at[pl.ds(c * ch, ch)]], emb_v, sem_e)
            cp1.wait()
            pltpu.sync_copy(emb_v, xemb_out.at[pl.ds(base + c * ch, ch)])
        cp2.wait()
        cp3.wait()
        pltpu.sync_copy(cent_v, cent_out.at[pl.ds(base, tpw)])
        pltpu.sync_copy(crows_v, crows_out.at[pl.ds(cbase, cpw)])

    fn = pl.kernel(
        body,
        out_type=(jax.ShapeDtypeStruct((T, PADW), F32),
                  jax.ShapeDtypeStruct((T,), jnp.int32),
                  jax.ShapeDtypeStruct((NLOC_CAND, PADW), F32)),
        mesh=mesh,
        scratch_types=[pltpu.VMEM((tpw,), jnp.int32),
                       pltpu.VMEM((ch, PADW), F32),
                       pltpu.VMEM((tpw,), jnp.int32),
                       pltpu.VMEM((cpw,), jnp.int32),
                       pltpu.VMEM((cpw, PADW), F32),
                       pltpu.SemaphoreType.DMA,
                       pltpu.SemaphoreType.DMA,
                       pltpu.SemaphoreType.DMA],
    )
    return fn(vecs_pad, xv, i_arr, cand_flat)


def _tables_body(cr, te, w1, b1, w2, b2, out, ns_out):
    v = cr[...]
    t2 = te[2:3, :]
    h = _mdot(v, w1[0:D, :]) + _mdot(t2, w1[D:2 * D, :]) + b1[...]
    h = jnp.maximum(h, 0.0)
    s = _mdot(h, w2[...]) + b2[...]
    out[...] = s
    ns_out[...] = jnp.sum(s * s, axis=1, keepdims=True)


def _tc_tables(cand_rows, te, wi1, bi1, wi2, bi2):
    return pl.pallas_call(
        _tables_body,
        out_shape=(jax.ShapeDtypeStruct((NLOC_CAND, D), F32),
                   jax.ShapeDtypeStruct((NLOC_CAND, 1), F32)),
    )(cand_rows, te, wi1, bi1.reshape(1, -1), wi2, bi2.reshape(1, -1))


_EULER = math.e


def _main_body(x4_ref, x3_ref, x2_ref, x1_ref, xe_ref, ts_ref, ct_ref,
               s2_ref, ns_ref, r2_ref,
               ws1, bs1, ws2, bs2, wo1, bo1, wo2, bo2, te, out_ref):
    xe = xe_ref[0][:, 0:D]
    tsl = ts_ref[0, 0]
    ct = ct_ref[0, 0]

    hacc = (_mdot(x4_ref[0][:, 0:D], ws1[0:D, :])
            + _mdot(x3_ref[0][:, 0:D], ws1[D:2 * D, :])
            + _mdot(x2_ref[0][:, 0:D], ws1[2 * D:3 * D, :])
            + _mdot(x1_ref[0][:, 0:D], ws1[3 * D:4 * D, :])
            + _mdot(xe, ws1[4 * D:5 * D, :]))
    h1 = jnp.maximum(hacc + bs1[...], 0.0)
    hseq = _mdot(h1, ws2[...]) + bs2[...]

    hh = tsl % 24
    m0 = (hh >= 22) | (hh < 6)
    m1 = (hh >= 6) & (hh < 14)
    m2 = jnp.logical_not(m0 | m1)
    tep = _mdot(te[0:3, :], wo1[D:2 * D, :])
    tec = (m0.astype(F32)[:, None] * tep[0:1, :]
           + m1.astype(F32)[:, None] * tep[1:2, :]
           + m2.astype(F32)[:, None] * tep[2:3, :])
    x1 = jnp.maximum(_mdot(xe, wo1[0:D, :]) + tec + bo1[...], 0.0)
    xi = _mdot(x1, wo2[...]) + bo2[...]

    q = (hseq + xi) * 0.5

    iota64 = lax.broadcasted_iota(jnp.int32, (USERS, NCAND), 1)
    oh16 = (ct[:, None] == iota64).astype(BF16)
    cand_sd = _dot01B(oh16, s2_ref[...])
    nc = _dot01B(oh16, ns_ref[...])
    cand_r = _dot01B_2(oh16, r2_ref[...])

    qc = q[:, 0:1] * cand_sd[:, 0:NCAND]
    for dd in range(1, D):
        qc = qc + q[:, dd:dd + 1] * cand_sd[:, dd * NCAND:(dd + 1) * NCAND]
    q2 = jnp.sum(q * q, axis=1, keepdims=True)
    d2 = jnp.maximum(q2 - 2.0 * qc + nc, 0.0)
    score = jnp.exp(-0.02 * jnp.sqrt(d2 + 1e-12))

    neg = F32(-1e30)
    alive = jnp.ones((USERS, NCAND), dtype=jnp.bool_)
    sel = jnp.zeros((USERS, NCAND), dtype=jnp.bool_)
    for _ in range(10):
        cur = jnp.where(alive, score, neg)
        m = jnp.max(cur, axis=1, keepdims=True)
        first = jnp.min(jnp.where(cur == m, iota64, NCAND),
                        axis=1, keepdims=True)
        pick = iota64 == first
        sel = sel | pick
        alive = alive & jnp.logical_not(pick)

    wt = jnp.where(sel, jnp.exp(score), 0.0)
    z = jnp.sum(wt, axis=1, keepdims=True) + _EULER
    rexp16 = (lax.broadcasted_iota(jnp.int32, (NCAND, CD), 1) // D
              == lax.broadcasted_iota(jnp.int32, (NCAND, CD), 0)).astype(BF16)
    wrep = _dotA01_2(wt, rexp16)
    tile_t16 = (lax.broadcasted_iota(jnp.int32, (CD, D), 0) % D
                == lax.broadcasted_iota(jnp.int32, (CD, D), 1)).astype(BF16)
    outn = _dotA01_2(wrep * cand_r, tile_t16)
    out_ref[0] = (outn + _EULER * xe) / z


def _tc_main(xp3, ts3, ct3, s2, ns2, r2,
             ws1, bs1, ws2, bs2, wo1, bo1, wo2, bo2, te):
    bs1, bs2 = bs1.reshape(1, -1), bs2.reshape(1, -1)
    bo1, bo2 = bo1.reshape(1, -1), bo2.reshape(1, -1)
    full2 = lambda a: pl.BlockSpec(a.shape, lambda s: (0, 0))

    def shift_spec(k):
        return pl.BlockSpec(
            (1, USERS, PADW),
            lambda s, k=k: (jnp.where(s >= k, s - k, s), 0, 0))

    in_specs = [
        shift_spec(4), shift_spec(3), shift_spec(2), shift_spec(1),
        shift_spec(0),
        pl.BlockSpec((1, 1, USERS), lambda s: (s, 0, 0)),
        pl.BlockSpec((1, 1, USERS), lambda s: (s, 0, 0)),
        full2(s2), full2(ns2), full2(r2),
        full2(ws1), full2(bs1), full2(ws2), full2(bs2),
        full2(wo1), full2(bo1), full2(wo2), full2(bo2), full2(te),
    ]
    return pl.pallas_call(
        _main_body,
        grid=(SEQ,),
        in_specs=in_specs,
        out_specs=pl.BlockSpec((1, USERS, D), lambda s: (s, 0, 0)),
        out_shape=jax.ShapeDtypeStruct((SEQ, USERS, D), F32),
    )(xp3, xp3, xp3, xp3, xp3, ts3, ct3, s2, ns2, r2,
      ws1, bs1, ws2, bs2, wo1, bo1, wo2, bo2, te)


def kernel(x, t_slot, y, y_t_slot, vecs_use, I_array, cand_table,
           time_embeddings, Ws1, bs1, Ws2, bs2, Wo1, bo1, Wo2, bo2,
           Wi1, bi1, Wi2, bi2):
    xv = x.reshape(-1).astype(jnp.int32)
    i_arr = I_array.astype(jnp.int32)
    cf = cand_table.reshape(-1).astype(jnp.int32)
    vecs_pad = jnp.pad(vecs_use, ((0, 0), (0, PADW - D)))

    x_emb_pad, cent, cand_rows_pad = _sc_gather(vecs_pad, xv, i_arr, cf)
    cand_rows = cand_rows_pad[:, :D]

    s_tab, ns = _tc_tables(cand_rows, time_embeddings, Wi1, bi1, Wi2, bi2)
    s2 = s_tab.reshape(NCENT, NCAND, D).transpose(0, 2, 1).reshape(NCENT, CD)
    ns2 = ns.reshape(NCENT, NCAND)
    r2 = cand_rows.reshape(NCENT, CD)

    xp3 = x_emb_pad.reshape(SEQ, USERS, PADW)
    ts3 = t_slot.reshape(SEQ, 1, USERS).astype(jnp.int32)
    ct3 = cent.reshape(SEQ, 1, USERS)

    out = _tc_main(xp3, ts3, ct3, s2, ns2, r2,
                   Ws1, bs1, Ws2, bs2, Wo1, bo1, Wo2, bo2, time_embeddings)
    return out.reshape(T, D)


# speedup vs baseline: 33.3315x; 1.5663x over previous
import math

import jax
import jax.numpy as jnp
from jax import lax
from jax.experimental import pallas as pl
from jax.experimental.pallas import tpu as pltpu
from jax.experimental.pallas import tpu_sc as plsc

F32 = jnp.float32
SEQ, USERS = 50, 1024
T = SEQ * USERS
D = 20
NCENT = 64
NCAND = 64
CD = NCAND * D
NLOC_CAND = NCENT * NCAND
PREC = lax.Precision.HIGHEST


def _dot(a, b):
    return jnp.dot(a, b, precision=PREC)


BF16 = jnp.bfloat16


def _bdot(a16, b16):
    return jnp.dot(a16, b16, preferred_element_type=F32)


def _split3(x):
    h1 = x.astype(BF16)
    r1 = x - h1.astype(F32)
    h2 = r1.astype(BF16)
    r2 = r1 - h2.astype(F32)
    h3 = r2.astype(BF16)
    return h1, h2, h3


def _dotA01(a, b16):
    return sum(_bdot(h, b16) for h in _split3(a))


def _dot01B(a16, b):
    return sum(_bdot(a16, h) for h in _split3(b))


def _split2(x):
    h1 = x.astype(BF16)
    h2 = (x - h1.astype(F32)).astype(BF16)
    return h1, h2


def _dotA01_2(a, b16):
    return sum(_bdot(h, b16) for h in _split2(a))


def _dot01B_2(a16, b):
    return sum(_bdot(a16, h) for h in _split2(b))


def _mdot(a, b):
    return jnp.dot(a.astype(jnp.bfloat16), b.astype(jnp.bfloat16),
                   preferred_element_type=jnp.float32)


PADW = 128


def _sc_gather(vecs_pad, xv, i_arr, cand_flat):
    info = plsc.get_sparse_core_info()
    ncores, nsub = info.num_cores, info.num_subcores
    nw = ncores * nsub
    tpw = T // nw
    cpw = NLOC_CAND // nw
    ch = 400
    mesh = plsc.VectorSubcoreMesh(core_axis_name="c", subcore_axis_name="s")

    def body(vecs_hbm, xv_hbm, i_hbm, cf_hbm, xemb_out, cent_out, crows_out,
             idx_v, emb_v, cent_v, cidx_v, crows_v, sem_e, sem_c, sem_r):
        wid = lax.axis_index("s") * ncores + lax.axis_index("c")
        base = wid * tpw
        pltpu.sync_copy(xv_hbm.at[pl.ds(base, tpw)], idx_v)
        cp2 = pltpu.async_copy(i_hbm.at[idx_v], cent_v, sem_c)
        cbase = wid * cpw
        pltpu.sync_copy(cf_hbm.at[pl.ds(cbase, cpw)], cidx_v)
        cp3 = pltpu.async_copy(vecs_hbm.at[cidx_v], crows_v, sem_r)
        for c in range(tpw // ch):
            cp1 = pltpu.async_copy(
                vecs_hbm.at[idx_v.at[pl.ds(c * ch, ch)]], emb_v, sem_e)
            cp1.wait()
            pltpu.sync_copy(emb_v, xemb_out.at[pl.ds(base + c * ch, ch)])
        cp2.wait()
        cp3.wait()
        pltpu.sync_copy(cent_v, cent_out.at[pl.ds(base, tpw)])
        pltpu.sync_copy(crows_v, crows_out.at[pl.ds(cbase, cpw)])

    fn = pl.kernel(
        body,
        out_type=(jax.ShapeDtypeStruct((T, PADW), F32),
                  jax.ShapeDtypeStruct((T,), jnp.int32),
                  jax.ShapeDtypeStruct((NLOC_CAND, PADW), F32)),
        mesh=mesh,
        scratch_types=[pltpu.VMEM((tpw,), jnp.int32),
                       pltpu.VMEM((ch, PADW), F32),
                       pltpu.VMEM((tpw,), jnp.int32),
                       pltpu.VMEM((cpw,), jnp.int32),
                       pltpu.VMEM((cpw, PADW), F32),
                       pltpu.SemaphoreType.DMA,
                       pltpu.SemaphoreType.DMA,
                       pltpu.SemaphoreType.DMA],
    )
    return fn(vecs_pad, xv, i_arr, cand_flat)


def _tables_body(cr, te, w1, b1, w2, b2, out, ns_out):
    v = cr[...]
    t2 = te[2:3, :]
    h = _mdot(v, w1[0:D, :]) + _mdot(t2, w1[D:2 * D, :]) + b1[...]
    h = jnp.maximum(h, 0.0)
    s = _mdot(h, w2[...]) + b2[...]
    out[...] = s
    ns_out[...] = jnp.sum(s * s, axis=1, keepdims=True)


def _tc_tables(cand_rows, te, wi1, bi1, wi2, bi2):
    return pl.pallas_call(
        _tables_body,
        out_shape=(jax.ShapeDtypeStruct((NLOC_CAND, D), F32),
                   jax.ShapeDtypeStruct((NLOC_CAND, 1), F32)),
    )(cand_rows, te, wi1, bi1.reshape(1, -1), wi2, bi2.reshape(1, -1))


_EULER = math.e


def _main_body(x4_ref, x3_ref, x2_ref, x1_ref, xe_ref, ts_ref, ct_ref,
               s2_ref, ns_ref, r2_ref,
               ws1, bs1, ws2, bs2, wo1, bo1, wo2, bo2, te, out_ref):
    xe = xe_ref[0][:, 0:D]
    tsl = ts_ref[0, 0]
    ct = ct_ref[0, 0]

    hacc = (_mdot(x4_ref[0][:, 0:D], ws1[0:D, :])
            + _mdot(x3_ref[0][:, 0:D], ws1[D:2 * D, :])
            + _mdot(x2_ref[0][:, 0:D], ws1[2 * D:3 * D, :])
            + _mdot(x1_ref[0][:, 0:D], ws1[3 * D:4 * D, :])
            + _mdot(xe, ws1[4 * D:5 * D, :]))
    h1 = jnp.maximum(hacc + bs1[...], 0.0)
    hseq = _mdot(h1, ws2[...]) + bs2[...]

    hh = tsl % 24
    m0 = (hh >= 22) | (hh < 6)
    m1 = (hh >= 6) & (hh < 14)
    m2 = jnp.logical_not(m0 | m1)
    tep = _mdot(te[0:3, :], wo1[D:2 * D, :])
    tec = (m0.astype(F32)[:, None] * tep[0:1, :]
           + m1.astype(F32)[:, None] * tep[1:2, :]
           + m2.astype(F32)[:, None] * tep[2:3, :])
    x1 = jnp.maximum(_mdot(xe, wo1[0:D, :]) + tec + bo1[...], 0.0)
    xi = _mdot(x1, wo2[...]) + bo2[...]

    q = (hseq + xi) * 0.5

    iota64t = lax.broadcasted_iota(jnp.int32, (NCAND, USERS), 0)
    oh16 = (ct[None, :] == iota64t).astype(BF16)
    cand_sd = _dotA01(s2_ref[...], oh16)
    nc = _dotA01(ns_ref[...], oh16)
    cand_r = _dotA01_2(r2_ref[...], oh16)

    qt = jnp.transpose(q)
    qc = qt[0:1, :] * cand_sd[0:NCAND, :]
    for dd in range(1, D):
        qc = qc + qt[dd:dd + 1, :] * cand_sd[dd * NCAND:(dd + 1) * NCAND, :]
    q2 = jnp.sum(qt * qt, axis=0, keepdims=True)
    d2 = jnp.maximum(q2 - 2.0 * qc + nc, 0.0)
    score = jnp.exp(-0.02 * jnp.sqrt(d2 + 1e-12))

    neg = F32(-1e30)
    alive = jnp.ones((NCAND, USERS), dtype=jnp.bool_)
    sel = jnp.zeros((NCAND, USERS), dtype=jnp.bool_)
    for _ in range(10):
        cur = jnp.where(alive, score, neg)
        m = jnp.max(cur, axis=0, keepdims=True)
        first = jnp.min(jnp.where(cur == m, iota64t, NCAND),
                        axis=0, keepdims=True)
        pick = iota64t == first
        sel = sel | pick
        alive = alive & jnp.logical_not(pick)

    wt = jnp.where(sel, jnp.exp(score), 0.0)
    z = jnp.sum(wt, axis=0, keepdims=True) + _EULER
    rsum16 = (lax.broadcasted_iota(jnp.int32, (CD, NCAND), 0) // D
              == lax.broadcasted_iota(jnp.int32, (CD, NCAND), 1)).astype(BF16)
    wrep = sum(_bdot(rsum16, h) for h in _split2(wt))
    tile16 = (lax.broadcasted_iota(jnp.int32, (D, CD), 1) % D
              == lax.broadcasted_iota(jnp.int32, (D, CD), 0)).astype(BF16)
    outn = sum(_bdot(tile16, h) for h in _split2(wrep * cand_r))
    res_t = (outn + _EULER * jnp.transpose(xe)) / z
    out_ref[0] = jnp.transpose(res_t)


def _tc_main(xp3, ts3, ct3, s2, ns2, r2,
             ws1, bs1, ws2, bs2, wo1, bo1, wo2, bo2, te):
    bs1, bs2 = bs1.reshape(1, -1), bs2.reshape(1, -1)
    bo1, bo2 = bo1.reshape(1, -1), bo2.reshape(1, -1)
    full2 = lambda a: pl.BlockSpec(a.shape, lambda s: (0, 0))

    def shift_spec(k):
        return pl.BlockSpec(
            (1, USERS, PADW),
            lambda s, k=k: (jnp.where(s >= k, s - k, s), 0, 0))

    in_specs = [
        shift_spec(4), shift_spec(3), shift_spec(2), shift_spec(1),
        shift_spec(0),
        pl.BlockSpec((1, 1, USERS), lambda s: (s, 0, 0)),
        pl.BlockSpec((1, 1, USERS), lambda s: (s, 0, 0)),
        full2(s2), full2(ns2), full2(r2),
        full2(ws1), full2(bs1), full2(ws2), full2(bs2),
        full2(wo1), full2(bo1), full2(wo2), full2(bo2), full2(te),
    ]
    return pl.pallas_call(
        _main_body,
        grid=(SEQ,),
        in_specs=in_specs,
        out_specs=pl.BlockSpec((1, USERS, D), lambda s: (s, 0, 0)),
        out_shape=jax.ShapeDtypeStruct((SEQ, USERS, D), F32),
    )(xp3, xp3, xp3, xp3, xp3, ts3, ct3, s2, ns2, r2,
      ws1, bs1, ws2, bs2, wo1, bo1, wo2, bo2, te)


def kernel(x, t_slot, y, y_t_slot, vecs_use, I_array, cand_table,
           time_embeddings, Ws1, bs1, Ws2, bs2, Wo1, bo1, Wo2, bo2,
           Wi1, bi1, Wi2, bi2):
    xv = x.reshape(-1).astype(jnp.int32)
    i_arr = I_array.astype(jnp.int32)
    cf = cand_table.reshape(-1).astype(jnp.int32)
    vecs_pad = jnp.pad(vecs_use, ((0, 0), (0, PADW - D)))

    x_emb_pad, cent, cand_rows_pad = _sc_gather(vecs_pad, xv, i_arr, cf)
    cand_rows = cand_rows_pad[:, :D]

    s_tab, ns = _tc_tables(cand_rows, time_embeddings, Wi1, bi1, Wi2, bi2)
    s2 = s_tab.reshape(NCENT, NCAND, D).transpose(2, 1, 0).reshape(CD, NCENT)
    ns2 = ns.reshape(NCENT, NCAND).T
    r2 = cand_rows.reshape(NCENT, CD).T

    xp3 = x_emb_pad.reshape(SEQ, USERS, PADW)
    ts3 = t_slot.reshape(SEQ, 1, USERS).astype(jnp.int32)
    ct3 = cent.reshape(SEQ, 1, USERS)

    out = _tc_main(xp3, ts3, ct3, s2, ns2, r2,
                   Ws1, bs1, Ws2, bs2, Wo1, bo1, Wo2, bo2, time_embeddings)
    return out.reshape(T, D)


# speedup vs baseline: 33.3322x; 1.0000x over previous
import math

import jax
import jax.numpy as jnp
from jax import lax
from jax.experimental import pallas as pl
from jax.experimental.pallas import tpu as pltpu
from jax.experimental.pallas import tpu_sc as plsc

F32 = jnp.float32
SEQ, USERS = 50, 1024
T = SEQ * USERS
D = 20
NCENT = 64
NCAND = 64
CD = NCAND * D
NLOC_CAND = NCENT * NCAND
BF16 = jnp.bfloat16


def _bdot(a16, b16):
    return jnp.dot(a16, b16, preferred_element_type=F32)


def _split3(x):
    h1 = x.astype(BF16)
    r1 = x - h1.astype(F32)
    h2 = r1.astype(BF16)
    r2 = r1 - h2.astype(F32)
    h3 = r2.astype(BF16)
    return h1, h2, h3


def _dotA01(a, b16):
    return sum(_bdot(h, b16) for h in _split3(a))


def _dot01B(a16, b):
    return sum(_bdot(a16, h) for h in _split3(b))


def _split2(x):
    h1 = x.astype(BF16)
    h2 = (x - h1.astype(F32)).astype(BF16)
    return h1, h2


def _dotA01_2(a, b16):
    return sum(_bdot(h, b16) for h in _split2(a))


def _dot01B_2(a16, b):
    return sum(_bdot(a16, h) for h in _split2(b))


def _mdot(a, b):
    return jnp.dot(a.astype(jnp.bfloat16), b.astype(jnp.bfloat16),
                   preferred_element_type=jnp.float32)


PADW = 128


def _sc_gather(vecs_pad, xv, i_arr, cand_flat):
    info = plsc.get_sparse_core_info()
    ncores, nsub = info.num_cores, info.num_subcores
    nw = ncores * nsub
    tpw = T // nw
    cpw = NLOC_CAND // nw
    ch = 400
    mesh = plsc.VectorSubcoreMesh(core_axis_name="c", subcore_axis_name="s")

    def body(vecs_hbm, xv_hbm, i_hbm, cf_hbm, xemb_out, cent_out, crows_out,
             idx_v, emb_v, cent_v, cidx_v, crows_v, sem_e, sem_c, sem_r):
        wid = lax.axis_index("s") * ncores + lax.axis_index("c")
        base = wid * tpw
        pltpu.sync_copy(xv_hbm.at[pl.ds(base, tpw)], idx_v)
        cp2 = pltpu.async_copy(i_hbm.at[idx_v], cent_v, sem_c)
        cbase = wid * cpw
        pltpu.sync_copy(cf_hbm.at[pl.ds(cbase, cpw)], cidx_v)
        cp3 = pltpu.async_copy(vecs_hbm.at[cidx_v], crows_v, sem_r)
        for c in range(tpw // ch):
            cp1 = pltpu.async_copy(
                vecs_hbm.at[idx_v.at[pl.ds(c * ch, ch)]], emb_v, sem_e)
            cp1.wait()
            pltpu.sync_copy(emb_v, xemb_out.at[pl.ds(base + c * ch, ch)])
        cp2.wait()
        cp3.wait()
        pltpu.sync_copy(cent_v, cent_out.at[pl.ds(base, tpw)])
        pltpu.sync_copy(crows_v, crows_out.at[pl.ds(cbase, cpw)])

    fn = pl.kernel(
        body,
        out_type=(jax.ShapeDtypeStruct((T, PADW), F32),
                  jax.ShapeDtypeStruct((T,), jnp.int32),
                  jax.ShapeDtypeStruct((NLOC_CAND, PADW), F32)),
        mesh=mesh,
        scratch_types=[pltpu.VMEM((tpw,), jnp.int32),
                       pltpu.VMEM((ch, PADW), F32),
                       pltpu.VMEM((tpw,), jnp.int32),
                       pltpu.VMEM((cpw,), jnp.int32),
                       pltpu.VMEM((cpw, PADW), F32),
                       pltpu.SemaphoreType.DMA,
                       pltpu.SemaphoreType.DMA,
                       pltpu.SemaphoreType.DMA],
    )
    return fn(vecs_pad, xv, i_arr, cand_flat)


def _tables_body(cr, te, w1, b1, w2, b2, out, ns_out):
    v = cr[...]
    t2 = te[2:3, :]
    h = _mdot(v, w1[0:D, :]) + _mdot(t2, w1[D:2 * D, :]) + b1[...]
    h = jnp.maximum(h, 0.0)
    s = _mdot(h, w2[...]) + b2[...]
    out[...] = s
    ns_out[...] = jnp.sum(s * s, axis=1, keepdims=True)


def _tc_tables(cand_rows, te, wi1, bi1, wi2, bi2):
    return pl.pallas_call(
        _tables_body,
        out_shape=(jax.ShapeDtypeStruct((NLOC_CAND, D), F32),
                   jax.ShapeDtypeStruct((NLOC_CAND, 1), F32)),
    )(cand_rows, te, wi1, bi1.reshape(1, -1), wi2, bi2.reshape(1, -1))


_EULER = math.e


def _main_body(x4_ref, x3_ref, x2_ref, x1_ref, xe_ref, ts_ref, ct_ref,
               s2_ref, ns_ref, r2_ref,
               ws1, bs1, ws2, bs2, wo1, bo1, wo2, bo2, te, out_ref):
    xe = xe_ref[0][:, 0:D]
    tsl = ts_ref[0, 0]
    ct = ct_ref[0, 0]

    hacc = (_mdot(x4_ref[0][:, 0:D], ws1[0:D, :])
            + _mdot(x3_ref[0][:, 0:D], ws1[D:2 * D, :])
            + _mdot(x2_ref[0][:, 0:D], ws1[2 * D:3 * D, :])
            + _mdot(x1_ref[0][:, 0:D], ws1[3 * D:4 * D, :])
            + _mdot(xe, ws1[4 * D:5 * D, :]))
    h1 = jnp.maximum(hacc + bs1[...], 0.0)
    hseq = _mdot(h1, ws2[...]) + bs2[...]

    hh = tsl % 24
    m0 = (hh >= 22) | (hh < 6)
    m1 = (hh >= 6) & (hh < 14)
    m2 = jnp.logical_not(m0 | m1)
    tep = _mdot(te[0:3, :], wo1[D:2 * D, :])
    tec = (m0.astype(F32)[:, None] * tep[0:1, :]
           + m1.astype(F32)[:, None] * tep[1:2, :]
           + m2.astype(F32)[:, None] * tep[2:3, :])
    x1 = jnp.maximum(_mdot(xe, wo1[0:D, :]) + tec + bo1[...], 0.0)
    xi = _mdot(x1, wo2[...]) + bo2[...]

    q = (hseq + xi) * 0.5

    iota64t = lax.broadcasted_iota(jnp.int32, (NCAND, USERS), 0)
    oh16 = (ct[None, :] == iota64t).astype(BF16)
    cand_sd = _dotA01(s2_ref[...], oh16)
    nc = _dotA01(ns_ref[...], oh16)
    cand_r = _dotA01_2(r2_ref[...], oh16)

    qt = jnp.transpose(q)
    qc = qt[0:1, :] * cand_sd[0:NCAND, :]
    for dd in range(1, D):
        qc = qc + qt[dd:dd + 1, :] * cand_sd[dd * NCAND:(dd + 1) * NCAND, :]
    q2 = jnp.sum(qt * qt, axis=0, keepdims=True)
    d2 = jnp.maximum(q2 - 2.0 * qc + nc, 0.0)
    score = jnp.exp(-0.02 * jnp.sqrt(d2 + 1e-12))

    neg = F32(-1e30)
    alive = jnp.ones((NCAND, USERS), dtype=jnp.bool_)
    sel = jnp.zeros((NCAND, USERS), dtype=jnp.bool_)
    for _ in range(10):
        cur = jnp.where(alive, score, neg)
        m = jnp.max(cur, axis=0, keepdims=True)
        first = jnp.min(jnp.where(cur == m, iota64t, NCAND),
                        axis=0, keepdims=True)
        pick = iota64t == first
        sel = sel | pick
        alive = alive & jnp.logical_not(pick)

    wt = jnp.where(sel, jnp.exp(score), 0.0)
    z = jnp.sum(wt, axis=0, keepdims=True) + _EULER
    rsum16 = (lax.broadcasted_iota(jnp.int32, (CD, NCAND), 0) // D
              == lax.broadcasted_iota(jnp.int32, (CD, NCAND), 1)).astype(BF16)
    wrep = sum(_bdot(rsum16, h) for h in _split2(wt))
    tile16 = (lax.broadcasted_iota(jnp.int32, (D, CD), 1) % D
              == lax.broadcasted_iota(jnp.int32, (D, CD), 0)).astype(BF16)
    outn = sum(_bdot(tile16, h) for h in _split2(wrep * cand_r))
    res_t = (outn + _EULER * jnp.transpose(xe)) / z
    out_ref[0] = jnp.transpose(res_t)


def _tc_main(xp3, ts3, ct3, s2, ns2, r2,
             ws1, bs1, ws2, bs2, wo1, bo1, wo2, bo2, te):
    bs1, bs2 = bs1.reshape(1, -1), bs2.reshape(1, -1)
    bo1, bo2 = bo1.reshape(1, -1), bo2.reshape(1, -1)
    full2 = lambda a: pl.BlockSpec(a.shape, lambda s: (0, 0))

    def shift_spec(k):
        return pl.BlockSpec(
            (1, USERS, PADW),
            lambda s, k=k: (jnp.where(s >= k, s - k, s), 0, 0))

    in_specs = [
        shift_spec(4), shift_spec(3), shift_spec(2), shift_spec(1),
        shift_spec(0),
        pl.BlockSpec((1, 1, USERS), lambda s: (s, 0, 0)),
        pl.BlockSpec((1, 1, USERS), lambda s: (s, 0, 0)),
        full2(s2), full2(ns2), full2(r2),
        full2(ws1), full2(bs1), full2(ws2), full2(bs2),
        full2(wo1), full2(bo1), full2(wo2), full2(bo2), full2(te),
    ]
    return pl.pallas_call(
        _main_body,
        grid=(SEQ,),
        in_specs=in_specs,
        out_specs=pl.BlockSpec((1, USERS, D), lambda s: (s, 0, 0)),
        out_shape=jax.ShapeDtypeStruct((SEQ, USERS, D), F32),
    )(xp3, xp3, xp3, xp3, xp3, ts3, ct3, s2, ns2, r2,
      ws1, bs1, ws2, bs2, wo1, bo1, wo2, bo2, te)


def kernel(x, t_slot, y, y_t_slot, vecs_use, I_array, cand_table,
           time_embeddings, Ws1, bs1, Ws2, bs2, Wo1, bo1, Wo2, bo2,
           Wi1, bi1, Wi2, bi2):
    xv = x.reshape(-1).astype(jnp.int32)
    i_arr = I_array.astype(jnp.int32)
    cf = cand_table.reshape(-1).astype(jnp.int32)
    vecs_pad = jnp.pad(vecs_use, ((0, 0), (0, PADW - D)))

    x_emb_pad, cent, cand_rows_pad = _sc_gather(vecs_pad, xv, i_arr, cf)
    cand_rows = cand_rows_pad[:, :D]

    s_tab, ns = _tc_tables(cand_rows, time_embeddings, Wi1, bi1, Wi2, bi2)
    s2 = s_tab.reshape(NCENT, NCAND, D).transpose(2, 1, 0).reshape(CD, NCENT)
    ns2 = ns.reshape(NCENT, NCAND).T
    r2 = cand_rows.reshape(NCENT, CD).T

    xp3 = x_emb_pad.reshape(SEQ, USERS, PADW)
    ts3 = t_slot.reshape(SEQ, 1, USERS).astype(jnp.int32)
    ct3 = cent.reshape(SEQ, 1, USERS)

    out = _tc_main(xp3, ts3, ct3, s2, ns2, r2,
                   Ws1, bs1, Ws2, bs2, Wo1, bo1, Wo2, bo2, time_embeddings)
    return out.reshape(T, D)
